# baseline (speedup 1.0000x reference)
import jax
import jax.numpy as jnp
from jax import lax
from jax.experimental import pallas as pl
from jax.experimental.pallas import tpu as pltpu

B, C, H, W = 2, 1024, 16, 16
E = 8
ND = 6
HF = int(C * 4.0)
N_TOK = B * H * W
HF_B = 2048
N_HFB = HF // HF_B

_EPS64 = 2.220446049250313e-16


def _routing(xf, prompt, de_cls, w_g, gate_boost, degra_W, degra_b):
    w1g = w_g[:C, :]
    w2g = w_g[C:, :]
    pbias = lax.dot_general(prompt, w2g, (((1,), (0,)), ((), ())),
                            preferred_element_type=jnp.float32)
    dbias = lax.dot_general(de_cls, degra_W, (((1,), (1,)), ((), ())),
                            preferred_element_type=jnp.float32)
    bias_b = pbias + gate_boost * (dbias + degra_b)
    logits = lax.dot_general(xf, w1g, (((1,), (0,)), ((), ())),
                             preferred_element_type=jnp.float32)
    row = lax.broadcasted_iota(jnp.int32, (N_TOK, E), 0)
    per_tok_bias = jnp.where(row < (N_TOK // B), bias_b[0:1, :], bias_b[1:2, :])
    logits = logits + per_tok_bias

    neg = jnp.float32(-jnp.inf)
    m1 = jnp.full((N_TOK, 1), neg, dtype=jnp.float32)
    m2 = jnp.full((N_TOK, 1), neg, dtype=jnp.float32)
    a1 = jnp.zeros((N_TOK, 1), dtype=jnp.int32)
    a2 = jnp.zeros((N_TOK, 1), dtype=jnp.int32)
    for j in range(E):
        lj = logits[:, j:j + 1]
        jn = jnp.int32(j)
        new1 = lj > m1
        new2 = jnp.logical_and(jnp.logical_not(new1), lj > m2)
        m2 = jnp.where(new1, m1, jnp.where(new2, lj, m2))
        a2 = jnp.where(new1, a1, jnp.where(new2, jn, a2))
        m1 = jnp.where(new1, lj, m1)
        a1 = jnp.where(new1, jn, a1)
    u = jnp.exp(m2 - m1)
    denom = 1.0 + u
    g1 = 1.0 / denom
    g2 = u / denom
    return a1, a2, g1, g2


def _balance_terms(vals):
    n = len(vals)
    s = vals[0]
    for v in vals[1:]:
        s = s + v
    m = s / n
    sq = (vals[0] - m) ** 2
    for v in vals[1:]:
        sq = sq + (v - m) ** 2
    var = sq / (n - 1)
    return var / (m * m + 1e-10)


def _gelu_fast(x):
    z = jnp.abs(x) * jnp.float32(0.7071067811865476)
    t = 1.0 / (1.0 + jnp.float32(0.3275911) * z)
    poly = t * (jnp.float32(0.254829592)
                + t * (jnp.float32(-0.284496736)
                       + t * (jnp.float32(1.421413741)
                              + t * (jnp.float32(-1.453152027)
                                     + t * jnp.float32(1.061405429)))))
    e = 1.0 - poly * jnp.exp(-z * z)
    erfv = jnp.where(x >= 0.0, e, -e)
    return 0.5 * x * (1.0 + erfv)


def _kernel(xf_ref, xb_ref, prompt_ref, de_cls_ref, w_g_ref, boost_ref,
            degW_ref, degb_ref, w1_ref, b1_ref, w2_ref, b2_ref,
            y_ref, loss_ref,
            a1_s, a2_s, g1_s, g2_s, outacc_s, ensacc_s):
    e = pl.program_id(0)
    h = pl.program_id(1)

    @pl.when(jnp.logical_and(e == 0, h == 0))
    def _do_routing():
        a1, a2, g1, g2 = _routing(
            xf_ref[...], prompt_ref[...], de_cls_ref[...], w_g_ref[...],
            boost_ref[0, 0], degW_ref[...], degb_ref[0, :])
        a1_s[...] = a1
        a2_s[...] = a2
        g1_s[...] = g1
        g2_s[...] = g2
        wv, sv = [], []
        for ee in range(E):
            ge = (jnp.where(a1 == ee, g1, 0.0) + jnp.where(a2 == ee, g2, 0.0))
            wv.append(jnp.sum(ge))
            sv.append(jnp.sum((ge > 0.0).astype(jnp.float32)))
        loss = _balance_terms(wv) + _balance_terms(sv)
        loss_ref[...] = jnp.reshape(loss, (1, 1))

    xb = xb_ref[...]
    w1b = w1_ref[0].astype(jnp.bfloat16)
    hid = lax.dot_general(xb, w1b, (((1,), (1,)), ((), ())),
                          preferred_element_type=jnp.float32)
    hid = hid + b1_ref[0]
    hid = _gelu_fast(hid)
    w2b = w2_ref[0].astype(jnp.bfloat16)
    contrib = lax.dot_general(hid.astype(jnp.bfloat16), w2b,
                              (((1,), (1,)), ((), ())),
                              preferred_element_type=jnp.float32)

    @pl.when(h == 0)
    def _init_out():
        outacc_s[...] = contrib

    @pl.when(h != 0)
    def _acc_out():
        outacc_s[...] = outacc_s[...] + contrib

    @pl.when(h == N_HFB - 1)
    def _combine():
        out = outacc_s[...] + b2_ref[0]
        gate = (jnp.where(a1_s[...] == e, g1_s[...], 0.0)
                + jnp.where(a2_s[...] == e, g2_s[...], 0.0))
        term = gate * jnp.exp(out)

        @pl.when(e == 0)
        def _():
            ensacc_s[...] = term

        @pl.when(e != 0)
        def _():
            ensacc_s[...] = ensacc_s[...] + term

        @pl.when(e == E - 1)
        def _final():
            ens = ensacc_s[...]
            ens = jnp.where(ens == 0.0, jnp.float32(_EPS64), ens)
            y_ref[...] = jnp.log(ens)


def kernel(x, prompt, de_cls, w_g, gate_boost, degra_W, degra_b,
           W1, b1, W2, b2):
    b, c, h, w_ = x.shape
    xf = jnp.transpose(x, (0, 2, 3, 1)).reshape(-1, c)
    boost = jnp.reshape(gate_boost, (1, 1)).astype(jnp.float32)
    degb = jnp.reshape(degra_b, (1, E))

    grid = (E, N_HFB)
    y, loss = pl.pallas_call(
        _kernel,
        grid=grid,
        in_specs=[
            pl.BlockSpec((N_TOK, C), lambda e, hh: (0, 0)),
            pl.BlockSpec((N_TOK, C), lambda e, hh: (0, 0)),
            pl.BlockSpec((B, C), lambda e, hh: (0, 0)),
            pl.BlockSpec((B, ND), lambda e, hh: (0, 0)),
            pl.BlockSpec((2 * C, E), lambda e, hh: (0, 0)),
            pl.BlockSpec((1, 1), lambda e, hh: (0, 0)),
            pl.BlockSpec((E, ND), lambda e, hh: (0, 0)),
            pl.BlockSpec((1, E), lambda e, hh: (0, 0)),
            pl.BlockSpec((1, HF_B, C), lambda e, hh: (e, hh, 0)),
            pl.BlockSpec((1, 1, HF_B), lambda e, hh: (e, 0, hh)),
            pl.BlockSpec((1, C, HF_B), lambda e, hh: (e, 0, hh)),
            pl.BlockSpec((1, 1, C), lambda e, hh: (e, 0, 0)),
        ],
        out_specs=[
            pl.BlockSpec((N_TOK, C), lambda e, hh: (0, 0)),
            pl.BlockSpec((1, 1), lambda e, hh: (0, 0)),
        ],
        out_shape=[
            jax.ShapeDtypeStruct((N_TOK, C), jnp.float32),
            jax.ShapeDtypeStruct((1, 1), jnp.float32),
        ],
        scratch_shapes=[
            pltpu.VMEM((N_TOK, 1), jnp.int32),
            pltpu.VMEM((N_TOK, 1), jnp.int32),
            pltpu.VMEM((N_TOK, 1), jnp.float32),
            pltpu.VMEM((N_TOK, 1), jnp.float32),
            pltpu.VMEM((N_TOK, C), jnp.float32),
            pltpu.VMEM((N_TOK, C), jnp.float32),
        ],
        compiler_params=pltpu.CompilerParams(
            dimension_semantics=("arbitrary", "arbitrary"),
        ),
    )(xf, xf.astype(jnp.bfloat16), prompt, de_cls, w_g, boost, degra_W, degb,
      W1, b1.reshape(E, 1, HF), W2, b2.reshape(E, 1, C))

    y = y.reshape(b, h, w_, c).transpose(0, 3, 1, 2)
    return y, jnp.reshape(loss, ())


# speedup vs baseline: 1.1149x; 1.1149x over previous
import jax
import jax.numpy as jnp
from jax import lax
from jax.experimental import pallas as pl
from jax.experimental.pallas import tpu as pltpu

B, C, H, W = 2, 1024, 16, 16
E = 8
ND = 6
HF = int(C * 4.0)
N_TOK = B * H * W
HF_B = 1024
N_HFB = HF // HF_B
NBLK = E * N_HFB
NSTEP = NBLK + 1

_EPS64 = 2.220446049250313e-16


def _routing(xf, prompt, de_cls, w_g, gate_boost, degra_W, degra_b):
    w1g = w_g[:C, :]
    w2g = w_g[C:, :]
    pbias = lax.dot_general(prompt, w2g, (((1,), (0,)), ((), ())),
                            preferred_element_type=jnp.float32)
    dbias = lax.dot_general(de_cls, degra_W, (((1,), (1,)), ((), ())),
                            preferred_element_type=jnp.float32)
    bias_b = pbias + gate_boost * (dbias + degra_b)
    logits = lax.dot_general(xf, w1g, (((1,), (0,)), ((), ())),
                             preferred_element_type=jnp.float32)
    row = lax.broadcasted_iota(jnp.int32, (N_TOK, E), 0)
    per_tok_bias = jnp.where(row < (N_TOK // B), bias_b[0:1, :], bias_b[1:2, :])
    logits = logits + per_tok_bias

    neg = jnp.float32(-jnp.inf)
    m1 = jnp.full((N_TOK, 1), neg, dtype=jnp.float32)
    m2 = jnp.full((N_TOK, 1), neg, dtype=jnp.float32)
    a1 = jnp.zeros((N_TOK, 1), dtype=jnp.int32)
    a2 = jnp.zeros((N_TOK, 1), dtype=jnp.int32)
    for j in range(E):
        lj = logits[:, j:j + 1]
        jn = jnp.int32(j)
        new1 = lj > m1
        new2 = jnp.logical_and(jnp.logical_not(new1), lj > m2)
        m2 = jnp.where(new1, m1, jnp.where(new2, lj, m2))
        a2 = jnp.where(new1, a1, jnp.where(new2, jn, a2))
        m1 = jnp.where(new1, lj, m1)
        a1 = jnp.where(new1, jn, a1)
    u = jnp.exp(m2 - m1)
    denom = 1.0 + u
    g1 = 1.0 / denom
    g2 = u / denom
    return a1, a2, g1, g2


def _balance_terms(vals):
    n = len(vals)
    s = vals[0]
    for v in vals[1:]:
        s = s + v
    m = s / n
    sq = (vals[0] - m) ** 2
    for v in vals[1:]:
        sq = sq + (v - m) ** 2
    var = sq / (n - 1)
    return var / (m * m + 1e-10)


def _kernel(xf_ref, xb_ref, prompt_ref, de_cls_ref, w_g_ref, boost_ref,
            degW_ref, degb_ref, w1_ref, b1_ref, w2_ref, b2_ref,
            y_ref, loss_ref,
            a1_s, a2_s, g1_s, g2_s, ghid_s, outacc_s, ensacc_s):
    k = pl.program_id(0)

    @pl.when(k == 0)
    def _do_routing():
        a1, a2, g1, g2 = _routing(
            xf_ref[...], prompt_ref[...], de_cls_ref[...], w_g_ref[...],
            boost_ref[0, 0], degW_ref[...], degb_ref[0, :])
        a1_s[...] = a1
        a2_s[...] = a2
        g1_s[...] = g1
        g2_s[...] = g2
        wv, sv = [], []
        for ee in range(E):
            ge = (jnp.where(a1 == ee, g1, 0.0) + jnp.where(a2 == ee, g2, 0.0))
            wv.append(jnp.sum(ge))
            sv.append(jnp.sum((ge > 0.0).astype(jnp.float32)))
        loss = _balance_terms(wv) + _balance_terms(sv)
        loss_ref[...] = jnp.reshape(loss, (1, 1))

    @pl.when(k > 0)
    def _stage_b():
        pk = k - 1
        h_prev = pk % N_HFB
        e_prev = pk // N_HFB
        prow = (pk % 2) * N_TOK
        ghid = ghid_s[pl.ds(prow, N_TOK), :]
        w2b = w2_ref[0]
        contrib = lax.dot_general(ghid, w2b.astype(jnp.bfloat16),
                                  (((1,), (1,)), ((), ())),
                                  preferred_element_type=jnp.float32)

        @pl.when(h_prev == 0)
        def _():
            outacc_s[...] = contrib

        @pl.when(h_prev != 0)
        def _():
            outacc_s[...] = outacc_s[...] + contrib

        @pl.when(h_prev == N_HFB - 1)
        def _expert_done():
            out = outacc_s[...] + b2_ref[0]
            gate = (jnp.where(a1_s[...] == e_prev, g1_s[...], 0.0)
                    + jnp.where(a2_s[...] == e_prev, g2_s[...], 0.0))
            term = gate * jnp.exp(out)

            @pl.when(e_prev == 0)
            def _():
                ensacc_s[...] = term

            @pl.when(e_prev != 0)
            def _():
                ensacc_s[...] = ensacc_s[...] + term

            @pl.when(e_prev == E - 1)
            def _final():
                ens = ensacc_s[...]
                ens = jnp.where(ens == 0.0, jnp.float32(_EPS64), ens)
                y_ref[...] = jnp.log(ens)

    @pl.when(k < NBLK)
    def _stage_a():
        xb = xb_ref[...]
        w1b = w1_ref[0]
        hid = lax.dot_general(xb, w1b.astype(jnp.bfloat16),
                              (((1,), (1,)), ((), ())),
                              preferred_element_type=jnp.float32)
        hid = hid + b1_ref[0]
        hid = 0.5 * hid * (1.0 + lax.erf(hid * jnp.float32(0.7071067811865476)))
        crow = (k % 2) * N_TOK
        ghid_s[pl.ds(crow, N_TOK), :] = hid.astype(jnp.bfloat16)


def _w1_map(k):
    kk = jnp.minimum(k, NBLK - 1)
    return (kk // N_HFB, kk % N_HFB, 0)


def _b1_map(k):
    kk = jnp.minimum(k, NBLK - 1)
    return (kk // N_HFB, 0, kk % N_HFB)


def _w2_map(k):
    pk = jnp.maximum(k - 1, 0)
    return (pk // N_HFB, 0, pk % N_HFB)


def _b2_map(k):
    pk = jnp.maximum(k - 1, 0)
    return (pk // N_HFB, 0, 0)


def kernel(x, prompt, de_cls, w_g, gate_boost, degra_W, degra_b,
           W1, b1, W2, b2):
    b, c, h, w_ = x.shape
    xf = jnp.transpose(x, (0, 2, 3, 1)).reshape(-1, c)
    boost = jnp.reshape(gate_boost, (1, 1)).astype(jnp.float32)
    degb = jnp.reshape(degra_b, (1, E))

    y, loss = pl.pallas_call(
        _kernel,
        grid=(NSTEP,),
        in_specs=[
            pl.BlockSpec((N_TOK, C), lambda k: (0, 0)),
            pl.BlockSpec((N_TOK, C), lambda k: (0, 0)),
            pl.BlockSpec((B, C), lambda k: (0, 0)),
            pl.BlockSpec((B, ND), lambda k: (0, 0)),
            pl.BlockSpec((2 * C, E), lambda k: (0, 0)),
            pl.BlockSpec((1, 1), lambda k: (0, 0)),
            pl.BlockSpec((E, ND), lambda k: (0, 0)),
            pl.BlockSpec((1, E), lambda k: (0, 0)),
            pl.BlockSpec((1, HF_B, C), _w1_map),
            pl.BlockSpec((1, 1, HF_B), _b1_map),
            pl.BlockSpec((1, C, HF_B), _w2_map),
            pl.BlockSpec((1, 1, C), _b2_map),
        ],
        out_specs=[
            pl.BlockSpec((N_TOK, C), lambda k: (0, 0)),
            pl.BlockSpec((1, 1), lambda k: (0, 0)),
        ],
        out_shape=[
            jax.ShapeDtypeStruct((N_TOK, C), jnp.float32),
            jax.ShapeDtypeStruct((1, 1), jnp.float32),
        ],
        scratch_shapes=[
            pltpu.VMEM((N_TOK, 1), jnp.int32),
            pltpu.VMEM((N_TOK, 1), jnp.int32),
            pltpu.VMEM((N_TOK, 1), jnp.float32),
            pltpu.VMEM((N_TOK, 1), jnp.float32),
            pltpu.VMEM((2 * N_TOK, HF_B), jnp.bfloat16),
            pltpu.VMEM((N_TOK, C), jnp.float32),
            pltpu.VMEM((N_TOK, C), jnp.float32),
        ],
        compiler_params=pltpu.CompilerParams(
            dimension_semantics=("arbitrary",),
        ),
    )(xf, xf.astype(jnp.bfloat16), prompt, de_cls, w_g, boost, degra_W, degb,
      W1, b1.reshape(E, 1, HF), W2, b2.reshape(E, 1, C))

    y = y.reshape(b, h, w_, c).transpose(0, 3, 1, 2)
    return y, jnp.reshape(loss, ())


# speedup vs baseline: 1.2004x; 1.0767x over previous
import jax
import jax.numpy as jnp
from jax import lax
from jax.experimental import pallas as pl
from jax.experimental.pallas import tpu as pltpu

B, C, H, W = 2, 1024, 16, 16
E = 8
ND = 6
HF = int(C * 4.0)
N_TOK = B * H * W
HF_B = 2048
N_HFB = HF // HF_B
NBLK = E * N_HFB
NSTEP = NBLK + 1

_EPS64 = 2.220446049250313e-16


def _routing(xf, prompt, de_cls, w_g, gate_boost, degra_W, degra_b):
    w1g = w_g[:C, :]
    w2g = w_g[C:, :]
    pbias = lax.dot_general(prompt, w2g, (((1,), (0,)), ((), ())),
                            preferred_element_type=jnp.float32)
    dbias = lax.dot_general(de_cls, degra_W, (((1,), (1,)), ((), ())),
                            preferred_element_type=jnp.float32)
    bias_b = pbias + gate_boost * (dbias + degra_b)
    logits = lax.dot_general(xf, w1g, (((1,), (0,)), ((), ())),
                             preferred_element_type=jnp.float32)
    row = lax.broadcasted_iota(jnp.int32, (N_TOK, E), 0)
    per_tok_bias = jnp.where(row < (N_TOK // B), bias_b[0:1, :], bias_b[1:2, :])
    logits = logits + per_tok_bias

    neg = jnp.float32(-jnp.inf)
    m1 = jnp.full((N_TOK, 1), neg, dtype=jnp.float32)
    m2 = jnp.full((N_TOK, 1), neg, dtype=jnp.float32)
    a1 = jnp.zeros((N_TOK, 1), dtype=jnp.int32)
    a2 = jnp.zeros((N_TOK, 1), dtype=jnp.int32)
    for j in range(E):
        lj = logits[:, j:j + 1]
        jn = jnp.int32(j)
        new1 = lj > m1
        new2 = jnp.logical_and(jnp.logical_not(new1), lj > m2)
        m2 = jnp.where(new1, m1, jnp.where(new2, lj, m2))
        a2 = jnp.where(new1, a1, jnp.where(new2, jn, a2))
        m1 = jnp.where(new1, lj, m1)
        a1 = jnp.where(new1, jn, a1)
    u = jnp.exp(m2 - m1)
    denom = 1.0 + u
    g1 = 1.0 / denom
    g2 = u / denom
    return a1, a2, g1, g2


def _balance_terms(vals):
    n = len(vals)
    s = vals[0]
    for v in vals[1:]:
        s = s + v
    m = s / n
    sq = (vals[0] - m) ** 2
    for v in vals[1:]:
        sq = sq + (v - m) ** 2
    var = sq / (n - 1)
    return var / (m * m + 1e-10)


def _kernel(xf_ref, xb_ref, prompt_ref, de_cls_ref, w_g_ref, boost_ref,
            degW_ref, degb_ref, w1_ref, b1_ref, w2_ref, b2_ref,
            y_ref, loss_ref,
            a1_s, a2_s, g1_s, g2_s, ghid_s, outacc_s, ensacc_s):
    k = pl.program_id(0)

    @pl.when(k == 0)
    def _do_routing():
        a1, a2, g1, g2 = _routing(
            xf_ref[...], prompt_ref[...], de_cls_ref[...], w_g_ref[...],
            boost_ref[0, 0], degW_ref[...], degb_ref[0, :])
        a1_s[...] = a1
        a2_s[...] = a2
        g1_s[...] = g1
        g2_s[...] = g2
        wv, sv = [], []
        for ee in range(E):
            ge = (jnp.where(a1 == ee, g1, 0.0) + jnp.where(a2 == ee, g2, 0.0))
            wv.append(jnp.sum(ge))
            sv.append(jnp.sum((ge > 0.0).astype(jnp.float32)))
        loss = _balance_terms(wv) + _balance_terms(sv)
        loss_ref[...] = jnp.reshape(loss, (1, 1))

    @pl.when(k > 0)
    def _stage_b():
        pk = k - 1
        h_prev = pk % N_HFB
        e_prev = pk // N_HFB
        prow = (pk % 2) * N_TOK
        ghid = ghid_s[pl.ds(prow, N_TOK), :]
        w2b = w2_ref[0]
        contrib = lax.dot_general(ghid, w2b.astype(jnp.bfloat16),
                                  (((1,), (1,)), ((), ())),
                                  preferred_element_type=jnp.float32)

        @pl.when(h_prev == 0)
        def _():
            outacc_s[...] = contrib

        @pl.when(h_prev != 0)
        def _():
            outacc_s[...] = outacc_s[...] + contrib

        @pl.when(h_prev == N_HFB - 1)
        def _expert_done():
            out = outacc_s[...] + b2_ref[0]
            gate = (jnp.where(a1_s[...] == e_prev, g1_s[...], 0.0)
                    + jnp.where(a2_s[...] == e_prev, g2_s[...], 0.0))
            term = gate * jnp.exp(out)

            @pl.when(e_prev == 0)
            def _():
                ensacc_s[...] = term

            @pl.when(e_prev != 0)
            def _():
                ensacc_s[...] = ensacc_s[...] + term

            @pl.when(e_prev == E - 1)
            def _final():
                ens = ensacc_s[...]
                ens = jnp.where(ens == 0.0, jnp.float32(_EPS64), ens)
                y_ref[...] = jnp.log(ens)

    @pl.when(k < NBLK)
    def _stage_a():
        xb = xb_ref[...]
        w1b = w1_ref[0]
        hid = lax.dot_general(xb, w1b.astype(jnp.bfloat16),
                              (((1,), (1,)), ((), ())),
                              preferred_element_type=jnp.float32)
        hid = hid + b1_ref[0]
        hid = 0.5 * hid * (1.0 + lax.erf(hid * jnp.float32(0.7071067811865476)))
        crow = (k % 2) * N_TOK
        ghid_s[pl.ds(crow, N_TOK), :] = hid.astype(jnp.bfloat16)


def _w1_map(k):
    kk = jnp.minimum(k, NBLK - 1)
    return (kk // N_HFB, kk % N_HFB, 0)


def _b1_map(k):
    kk = jnp.minimum(k, NBLK - 1)
    return (kk // N_HFB, 0, kk % N_HFB)


def _w2_map(k):
    pk = jnp.maximum(k - 1, 0)
    return (pk // N_HFB, 0, pk % N_HFB)


def _b2_map(k):
    pk = jnp.maximum(k - 1, 0)
    return (pk // N_HFB, 0, 0)


def kernel(x, prompt, de_cls, w_g, gate_boost, degra_W, degra_b,
           W1, b1, W2, b2):
    b, c, h, w_ = x.shape
    xf = jnp.transpose(x, (0, 2, 3, 1)).reshape(-1, c)
    boost = jnp.reshape(gate_boost, (1, 1)).astype(jnp.float32)
    degb = jnp.reshape(degra_b, (1, E))

    y, loss = pl.pallas_call(
        _kernel,
        grid=(NSTEP,),
        in_specs=[
            pl.BlockSpec((N_TOK, C), lambda k: (0, 0)),
            pl.BlockSpec((N_TOK, C), lambda k: (0, 0)),
            pl.BlockSpec((B, C), lambda k: (0, 0)),
            pl.BlockSpec((B, ND), lambda k: (0, 0)),
            pl.BlockSpec((2 * C, E), lambda k: (0, 0)),
            pl.BlockSpec((1, 1), lambda k: (0, 0)),
            pl.BlockSpec((E, ND), lambda k: (0, 0)),
            pl.BlockSpec((1, E), lambda k: (0, 0)),
            pl.BlockSpec((1, HF_B, C), _w1_map),
            pl.BlockSpec((1, 1, HF_B), _b1_map),
            pl.BlockSpec((1, C, HF_B), _w2_map),
            pl.BlockSpec((1, 1, C), _b2_map),
        ],
        out_specs=[
            pl.BlockSpec((N_TOK, C), lambda k: (0, 0)),
            pl.BlockSpec((1, 1), lambda k: (0, 0)),
        ],
        out_shape=[
            jax.ShapeDtypeStruct((N_TOK, C), jnp.float32),
            jax.ShapeDtypeStruct((1, 1), jnp.float32),
        ],
        scratch_shapes=[
            pltpu.VMEM((N_TOK, 1), jnp.int32),
            pltpu.VMEM((N_TOK, 1), jnp.int32),
            pltpu.VMEM((N_TOK, 1), jnp.float32),
            pltpu.VMEM((N_TOK, 1), jnp.float32),
            pltpu.VMEM((2 * N_TOK, HF_B), jnp.bfloat16),
            pltpu.VMEM((N_TOK, C), jnp.float32),
            pltpu.VMEM((N_TOK, C), jnp.float32),
        ],
        compiler_params=pltpu.CompilerParams(
            dimension_semantics=("arbitrary",),
        ),
    )(xf, xf.astype(jnp.bfloat16), prompt, de_cls, w_g, boost, degra_W, degb,
      W1, b1.reshape(E, 1, HF), W2, b2.reshape(E, 1, C))

    y = y.reshape(b, h, w_, c).transpose(0, 3, 1, 2)
    return y, jnp.reshape(loss, ())


# speedup vs baseline: 1.3017x; 1.0844x over previous
import jax
import jax.numpy as jnp
from jax import lax
from jax.experimental import pallas as pl
from jax.experimental.pallas import tpu as pltpu

B, C, H, W = 2, 1024, 16, 16
E = 8
ND = 6
HF = int(C * 4.0)
N_TOK = B * H * W
HF_B = 2048
N_HFB = HF // HF_B

_EPS64 = 2.220446049250313e-16


def _routing(xf, prompt, de_cls, w_g, gate_boost, degra_W, degra_b):
    w1g = w_g[:C, :]
    w2g = w_g[C:, :]
    pbias = lax.dot_general(prompt, w2g, (((1,), (0,)), ((), ())),
                            preferred_element_type=jnp.float32)
    dbias = lax.dot_general(de_cls, degra_W, (((1,), (1,)), ((), ())),
                            preferred_element_type=jnp.float32)
    bias_b = pbias + gate_boost * (dbias + degra_b)
    logits = lax.dot_general(xf, w1g, (((1,), (0,)), ((), ())),
                             preferred_element_type=jnp.float32)
    row = lax.broadcasted_iota(jnp.int32, (N_TOK, E), 0)
    per_tok_bias = jnp.where(row < (N_TOK // B), bias_b[0:1, :], bias_b[1:2, :])
    logits = logits + per_tok_bias

    neg = jnp.float32(-jnp.inf)
    m1 = jnp.full((N_TOK, 1), neg, dtype=jnp.float32)
    m2 = jnp.full((N_TOK, 1), neg, dtype=jnp.float32)
    a1 = jnp.zeros((N_TOK, 1), dtype=jnp.int32)
    a2 = jnp.zeros((N_TOK, 1), dtype=jnp.int32)
    for j in range(E):
        lj = logits[:, j:j + 1]
        jn = jnp.int32(j)
        new1 = lj > m1
        new2 = jnp.logical_and(jnp.logical_not(new1), lj > m2)
        m2 = jnp.where(new1, m1, jnp.where(new2, lj, m2))
        a2 = jnp.where(new1, a1, jnp.where(new2, jn, a2))
        m1 = jnp.where(new1, lj, m1)
        a1 = jnp.where(new1, jn, a1)
    u = jnp.exp(m2 - m1)
    denom = 1.0 + u
    g1 = 1.0 / denom
    g2 = u / denom
    return a1, a2, g1, g2


def _balance_terms(vals):
    n = len(vals)
    s = vals[0]
    for v in vals[1:]:
        s = s + v
    m = s / n
    sq = (vals[0] - m) ** 2
    for v in vals[1:]:
        sq = sq + (v - m) ** 2
    var = sq / (n - 1)
    return var / (m * m + 1e-10)


def _kernel(xf_ref, prompt_ref, de_cls_ref, w_g_ref, boost_ref,
            degW_ref, degb_ref, w1_ref, b1_ref, w2_ref, b2_ref,
            y_ref, loss_ref,
            a1_s, a2_s, g1_s, g2_s, outacc_s, ensacc_s):
    e = pl.program_id(0)
    h = pl.program_id(1)

    @pl.when(jnp.logical_and(e == 0, h == 0))
    def _do_routing():
        a1, a2, g1, g2 = _routing(
            xf_ref[...], prompt_ref[...], de_cls_ref[...], w_g_ref[...],
            boost_ref[0, 0], degW_ref[...], degb_ref[0, :])
        a1_s[...] = a1
        a2_s[...] = a2
        g1_s[...] = g1
        g2_s[...] = g2
        wv, sv = [], []
        for ee in range(E):
            ge = (jnp.where(a1 == ee, g1, 0.0) + jnp.where(a2 == ee, g2, 0.0))
            wv.append(jnp.sum(ge))
            sv.append(jnp.sum((ge > 0.0).astype(jnp.float32)))
        loss = _balance_terms(wv) + _balance_terms(sv)
        loss_ref[...] = jnp.reshape(loss, (1, 1))

    xb = xf_ref[...]
    w1b = w1_ref[0]
    hid = lax.dot_general(xb, w1b, (((1,), (1,)), ((), ())),
                          preferred_element_type=jnp.float32)
    hid = hid + b1_ref[0]
    hid = 0.5 * hid * (1.0 + lax.erf(hid * jnp.float32(0.7071067811865476)))
    w2b = w2_ref[0]
    contrib = lax.dot_general(hid, w2b,
                              (((1,), (1,)), ((), ())),
                              preferred_element_type=jnp.float32)

    @pl.when(h == 0)
    def _init_out():
        outacc_s[...] = contrib

    @pl.when(h != 0)
    def _acc_out():
        outacc_s[...] = outacc_s[...] + contrib

    @pl.when(h == N_HFB - 1)
    def _combine():
        out = outacc_s[...] + b2_ref[0]
        gate = (jnp.where(a1_s[...] == e, g1_s[...], 0.0)
                + jnp.where(a2_s[...] == e, g2_s[...], 0.0))
        term = gate * jnp.exp(out)

        @pl.when(e == 0)
        def _():
            ensacc_s[...] = term

        @pl.when(e != 0)
        def _():
            ensacc_s[...] = ensacc_s[...] + term

        @pl.when(e == E - 1)
        def _final():
            ens = ensacc_s[...]
            ens = jnp.where(ens == 0.0, jnp.float32(_EPS64), ens)
            y_ref[...] = jnp.log(ens)


def kernel(x, prompt, de_cls, w_g, gate_boost, degra_W, degra_b,
           W1, b1, W2, b2):
    b, c, h, w_ = x.shape
    xf = jnp.transpose(x, (0, 2, 3, 1)).reshape(-1, c)
    boost = jnp.reshape(gate_boost, (1, 1)).astype(jnp.float32)
    degb = jnp.reshape(degra_b, (1, E))

    grid = (E, N_HFB)
    y, loss = pl.pallas_call(
        _kernel,
        grid=grid,
        in_specs=[
            pl.BlockSpec((N_TOK, C), lambda e, hh: (0, 0)),
            pl.BlockSpec((B, C), lambda e, hh: (0, 0)),
            pl.BlockSpec((B, ND), lambda e, hh: (0, 0)),
            pl.BlockSpec((2 * C, E), lambda e, hh: (0, 0)),
            pl.BlockSpec((1, 1), lambda e, hh: (0, 0)),
            pl.BlockSpec((E, ND), lambda e, hh: (0, 0)),
            pl.BlockSpec((1, E), lambda e, hh: (0, 0)),
            pl.BlockSpec((1, HF_B, C), lambda e, hh: (e, hh, 0)),
            pl.BlockSpec((1, 1, HF_B), lambda e, hh: (e, 0, hh)),
            pl.BlockSpec((1, C, HF_B), lambda e, hh: (e, 0, hh)),
            pl.BlockSpec((1, 1, C), lambda e, hh: (e, 0, 0)),
        ],
        out_specs=[
            pl.BlockSpec((N_TOK, C), lambda e, hh: (0, 0)),
            pl.BlockSpec((1, 1), lambda e, hh: (0, 0)),
        ],
        out_shape=[
            jax.ShapeDtypeStruct((N_TOK, C), jnp.float32),
            jax.ShapeDtypeStruct((1, 1), jnp.float32),
        ],
        scratch_shapes=[
            pltpu.VMEM((N_TOK, 1), jnp.int32),
            pltpu.VMEM((N_TOK, 1), jnp.int32),
            pltpu.VMEM((N_TOK, 1), jnp.float32),
            pltpu.VMEM((N_TOK, 1), jnp.float32),
            pltpu.VMEM((N_TOK, C), jnp.float32),
            pltpu.VMEM((N_TOK, C), jnp.float32),
        ],
        compiler_params=pltpu.CompilerParams(
            dimension_semantics=("arbitrary", "arbitrary"),
        ),
    )(xf, prompt, de_cls, w_g, boost, degra_W, degb,
      W1, b1.reshape(E, 1, HF), W2, b2.reshape(E, 1, C))

    y = y.reshape(b, h, w_, c).transpose(0, 3, 1, 2)
    return y, jnp.reshape(loss, ())
